# Initial kernel scaffold; baseline (speedup 1.0000x reference)
import jax
import jax.numpy as jnp
from jax.experimental import pallas as pl


def kernel(x, adj, weight, root_weight):
    raise NotImplementedError("write your pallas kernel here")


# speedup vs baseline: 1.0855x; 1.0855x over previous
import jax
import jax.numpy as jnp
from jax.experimental import pallas as pl
from jax.experimental.pallas import tpu as pltpu


def _gcn_kernel(x_ref, adj_ref, w_ref, wr_ref, out_ref, support_ref):
    i = pl.program_id(0)

    @pl.when(i == 0)
    def _():
        support_ref[...] = jnp.dot(
            x_ref[...], w_ref[...], preferred_element_type=jnp.float32
        )

    bm = out_ref.shape[0]
    x_blk = x_ref[pl.ds(i * bm, bm), :]
    acc = jnp.dot(adj_ref[...], support_ref[...], preferred_element_type=jnp.float32)
    acc = acc + jnp.dot(x_blk, wr_ref[...], preferred_element_type=jnp.float32)
    out_ref[...] = acc


def kernel(x, adj, weight, root_weight):
    n, d_in = x.shape
    d_out = weight.shape[1]
    bm = 400
    return pl.pallas_call(
        _gcn_kernel,
        grid=(n // bm,),
        in_specs=[
            pl.BlockSpec((n, d_in), lambda i: (0, 0)),
            pl.BlockSpec((bm, n), lambda i: (i, 0)),
            pl.BlockSpec((d_in, d_out), lambda i: (0, 0)),
            pl.BlockSpec((d_in, d_out), lambda i: (0, 0)),
        ],
        out_specs=pl.BlockSpec((bm, d_out), lambda i: (i, 0)),
        out_shape=jax.ShapeDtypeStruct((n, d_out), jnp.float32),
        scratch_shapes=[pltpu.VMEM((n, d_out), jnp.float32)],
    )(x, adj, weight, root_weight)
